# baseline (speedup 1.0000x reference)
import functools

import jax
import jax.numpy as jnp
from jax import lax
from jax.experimental import pallas as pl
from jax.experimental.pallas import tpu as pltpu
from jax.experimental.pallas import tpu_sc as plsc

VOCAB = 100000
EMB = 32
BATCH = 1024
CTX = 20

NC = 2
NS = 16
NW = NC * NS
B_PER_W = BATCH // NW
IDX_PER_W = B_PER_W * CTX
GCHUNK = 128
NCHUNK = IDX_PER_W // GCHUNK


def _sc_body(idx_hbm, table_hbm, bow_hbm, idx_v, rows_v, out_v, sem):
    wid = lax.axis_index("s") * NC + lax.axis_index("c")
    pltpu.sync_copy(idx_hbm.at[wid], idx_v)
    copies = [
        pltpu.async_copy(
            table_hbm.at[idx_v.at[j]], rows_v.at[pl.ds(j * GCHUNK, GCHUNK)], sem
        )
        for j in range(NCHUNK)
    ]
    for c in copies:
        c.wait()

    inv = jnp.float32(1.0 / CTX)

    def body(b, _):
        r0 = b * CTX
        acc0 = jnp.zeros((16,), jnp.float32)
        acc1 = jnp.zeros((16,), jnp.float32)
        for c in range(CTX):
            acc0 = acc0 + rows_v[r0 + c, pl.ds(0, 16)]
            acc1 = acc1 + rows_v[r0 + c, pl.ds(16, 16)]
        out_v[b, pl.ds(0, 16)] = acc0 * inv
        out_v[b, pl.ds(16, 16)] = acc1 * inv
        return 0

    lax.fori_loop(0, B_PER_W, body, 0)
    pltpu.sync_copy(out_v, bow_hbm.at[pl.ds(wid * B_PER_W, B_PER_W)])


_sc_gather_mean = functools.partial(
    pl.kernel,
    out_type=jax.ShapeDtypeStruct((BATCH, EMB), jnp.float32),
    scratch_types=[
        pltpu.VMEM((NCHUNK, GCHUNK), jnp.int32),
        pltpu.VMEM((IDX_PER_W, EMB), jnp.float32),
        pltpu.VMEM((B_PER_W, EMB), jnp.float32),
        pltpu.SemaphoreType.DMA,
    ],
    mesh=plsc.VectorSubcoreMesh(core_axis_name="c", subcore_axis_name="s"),
    compiler_params=pltpu.CompilerParams(use_tc_tiling_on_sc=False),
)(_sc_body)


MBLK = 8
NBUF = 8
NQ = 1
MGRID = BATCH // MBLK
QROWS = MBLK // NQ


def _mm_body(bow_ref, wt_ref, b_ref, out_hbm, obuf, sems):
    i = pl.program_id(0)
    s = lax.rem(i, NBUF)

    @pl.when(i >= NBUF)
    def _wait_slot():
        for q in range(NQ):
            pltpu.make_async_copy(
                obuf.at[s, pl.ds(q * QROWS, QROWS)],
                out_hbm.at[pl.ds((i - NBUF) * MBLK + q * QROWS, QROWS)],
                sems.at[s, q],
            ).wait()

    obuf[s] = (
        lax.dot_general(
            bow_ref[pl.ds(i * MBLK, MBLK), :],
            wt_ref[...],
            (((1,), (0,)), ((), ())),
            preferred_element_type=jnp.float32,
        )
        + b_ref[...]
    )

    for q in range(NQ):
        pltpu.make_async_copy(
            obuf.at[s, pl.ds(q * QROWS, QROWS)],
            out_hbm.at[pl.ds(i * MBLK + q * QROWS, QROWS)],
            sems.at[s, q],
        ).start(priority=q)

    @pl.when(i == MGRID - 1)
    def _drain():
        for k in range(NBUF):
            j = i - (NBUF - 1) + k
            sl = lax.rem(j, NBUF)
            for q in range(NQ):
                pltpu.make_async_copy(
                    obuf.at[sl, pl.ds(q * QROWS, QROWS)],
                    out_hbm.at[pl.ds(j * MBLK + q * QROWS, QROWS)],
                    sems.at[sl, q],
                ).wait()


def _mm_body_simple(bow_ref, wt_ref, b_ref, out_ref):
    out_ref[...] = (
        lax.dot_general(
            bow_ref[...],
            wt_ref[...],
            (((1,), (0,)), ((), ())),
            preferred_element_type=jnp.float32,
        )
        + b_ref[...]
    )


def _matmul(bow, Wt, b2d):
    return pl.pallas_call(
        _mm_body,
        grid=(MGRID,),
        in_specs=[
            pl.BlockSpec((BATCH, EMB), lambda i: (0, 0)),
            pl.BlockSpec((EMB, VOCAB), lambda i: (0, 0)),
            pl.BlockSpec((1, VOCAB), lambda i: (0, 0)),
        ],
        out_specs=pl.BlockSpec(memory_space=pl.ANY),
        out_shape=jax.ShapeDtypeStruct((BATCH, VOCAB), jnp.float32),
        scratch_shapes=[
            pltpu.VMEM((NBUF, MBLK, VOCAB), jnp.float32),
            pltpu.SemaphoreType.DMA((NBUF, NQ)),
        ],
    )(bow, Wt, b2d)


def kernel(X, emb_table, W, b):
    idx = X.astype(jnp.int32).reshape(NW, NCHUNK, GCHUNK)
    bow = _sc_gather_mean(idx, emb_table)
    return _matmul(bow, W.T, b.reshape(1, VOCAB))


# speedup vs baseline: 1.0125x; 1.0125x over previous
import functools

import jax
import jax.numpy as jnp
from jax import lax
from jax.experimental import pallas as pl
from jax.experimental.pallas import tpu as pltpu
from jax.experimental.pallas import tpu_sc as plsc

VOCAB = 100000
EMB = 32
BATCH = 1024
CTX = 20

NC = 2
NS = 16
NW = NC * NS
B_PER_W = BATCH // NW
IDX_PER_W = B_PER_W * CTX
GCHUNK = 128
NCHUNK = IDX_PER_W // GCHUNK


def _sc_body(idx_hbm, table_hbm, bow_hbm, idx_v, rows_v, out_v, sem):
    wid = lax.axis_index("s") * NC + lax.axis_index("c")
    pltpu.sync_copy(idx_hbm.at[wid], idx_v)
    copies = [
        pltpu.async_copy(
            table_hbm.at[idx_v.at[j]], rows_v.at[pl.ds(j * GCHUNK, GCHUNK)], sem
        )
        for j in range(NCHUNK)
    ]
    for c in copies:
        c.wait()

    inv = jnp.float32(1.0 / CTX)

    def body(b, _):
        r0 = b * CTX
        acc0 = jnp.zeros((16,), jnp.float32)
        acc1 = jnp.zeros((16,), jnp.float32)
        for c in range(CTX):
            acc0 = acc0 + rows_v[r0 + c, pl.ds(0, 16)]
            acc1 = acc1 + rows_v[r0 + c, pl.ds(16, 16)]
        out_v[b, pl.ds(0, 16)] = acc0 * inv
        out_v[b, pl.ds(16, 16)] = acc1 * inv
        return 0

    lax.fori_loop(0, B_PER_W, body, 0)
    pltpu.sync_copy(out_v, bow_hbm.at[pl.ds(wid * B_PER_W, B_PER_W)])


_sc_gather_mean = functools.partial(
    pl.kernel,
    out_type=jax.ShapeDtypeStruct((BATCH, EMB), jnp.float32),
    scratch_types=[
        pltpu.VMEM((NCHUNK, GCHUNK), jnp.int32),
        pltpu.VMEM((IDX_PER_W, EMB), jnp.float32),
        pltpu.VMEM((B_PER_W, EMB), jnp.float32),
        pltpu.SemaphoreType.DMA,
    ],
    mesh=plsc.VectorSubcoreMesh(core_axis_name="c", subcore_axis_name="s"),
    compiler_params=pltpu.CompilerParams(use_tc_tiling_on_sc=False),
)(_sc_body)


MBLK = 32
MGRID = BATCH // MBLK


def _mm_body_simple(bow_ref, wt_ref, b_ref, out_ref):
    out_ref[...] = (
        lax.dot_general(
            bow_ref[...],
            wt_ref[...],
            (((1,), (0,)), ((), ())),
            preferred_element_type=jnp.float32,
        )
        + b_ref[...]
    )


def _matmul(bow, Wt, b2d):
    return pl.pallas_call(
        _mm_body_simple,
        grid=(MGRID,),
        in_specs=[
            pl.BlockSpec((MBLK, EMB), lambda i: (i, 0)),
            pl.BlockSpec((EMB, VOCAB), lambda i: (0, 0)),
            pl.BlockSpec((1, VOCAB), lambda i: (0, 0)),
        ],
        out_specs=pl.BlockSpec((MBLK, VOCAB), lambda i: (i, 0)),
        out_shape=jax.ShapeDtypeStruct((BATCH, VOCAB), jnp.float32),
        compiler_params=pltpu.CompilerParams(
            dimension_semantics=("parallel",),
        ),
    )(bow, Wt, b2d)


def kernel(X, emb_table, W, b):
    idx = X.astype(jnp.int32).reshape(NW, NCHUNK, GCHUNK)
    bow = _sc_gather_mean(idx, emb_table)
    return _matmul(bow, W.T, b.reshape(1, VOCAB))
